# baseline (speedup 1.0000x reference)
import functools

import jax
import jax.numpy as jnp
from jax import lax
from jax.experimental import pallas as pl
from jax.experimental.pallas import tpu as pltpu
from jax.experimental.pallas import tpu_sc as plsc

N_FIELDS = 26
VOCAB = 100000
EMB_DIM = 64
BATCH = 4096

NC = 2
NS = 16
NW = NC * NS

B_FLAT = BATCH * N_FIELDS
PER_W = B_FLAT // NW
CHUNK = 128
N_CH = PER_W // CHUNK


def _gather_body(ids_hbm, table_hbm, offs_hbm, out_hbm, idx_v, offs_v, rows_v, sem):
    wid = lax.axis_index("s") * NC + lax.axis_index("c")
    base = wid * PER_W

    pltpu.sync_copy(offs_hbm, offs_v)
    pltpu.sync_copy(ids_hbm.at[pl.ds(base, PER_W)], idx_v)

    def add_body(i, carry):
        s = i * 16
        idx_v[pl.ds(s, 16)] = idx_v[pl.ds(s, 16)] + offs_v[pl.ds(s, 16)]
        return carry

    lax.fori_loop(0, PER_W // 16, add_body, 0)

    def chunk_body(j, carry):
        pltpu.async_copy(
            table_hbm.at[idx_v.at[pl.ds(j * CHUNK, CHUNK)]], rows_v, sem
        ).wait()
        pltpu.sync_copy(rows_v, out_hbm.at[pl.ds(base + j * CHUNK, CHUNK)])
        return carry

    lax.fori_loop(0, N_CH, chunk_body, 0)


@jax.jit
def kernel(prop_ids, tables):
    ids_flat = prop_ids.astype(jnp.int32).reshape(B_FLAT)
    table_flat = tables.reshape(N_FIELDS * VOCAB, EMB_DIM)
    offs = jnp.tile(jnp.arange(N_FIELDS, dtype=jnp.int32) * VOCAB, PER_W // N_FIELDS)

    mesh = plsc.VectorSubcoreMesh(
        core_axis_name="c", subcore_axis_name="s", num_cores=NC, num_subcores=NS
    )
    out = pl.kernel(
        _gather_body,
        out_type=jax.ShapeDtypeStruct((B_FLAT, EMB_DIM), jnp.float32),
        mesh=mesh,
        scratch_types=[
            pltpu.VMEM((PER_W,), jnp.int32),
            pltpu.VMEM((PER_W,), jnp.int32),
            pltpu.VMEM((CHUNK, EMB_DIM), jnp.float32),
            pltpu.SemaphoreType.DMA,
        ],
        compiler_params=pltpu.CompilerParams(use_tc_tiling_on_sc=False),
    )(ids_flat, table_flat, offs)
    return out.reshape(BATCH, N_FIELDS * EMB_DIM)
